# baseline (speedup 1.0000x reference)
import functools

import jax
import jax.numpy as jnp
import numpy as np
from jax import lax
from jax.experimental import pallas as pl
from jax.experimental.pallas import tpu as pltpu
from jax.experimental.pallas import tpu_sc as plsc

N_LEVELS = 16
F_PER_LEVEL = 2
LOG2_T = 19
T = 1 << LOG2_T
MASK = T - 1
BASE_RES = 16
PER_LEVEL_SCALE = 1.5
P1 = 2654435761
P2 = 805459861
P3 = 3674653429
HIDDEN = 64
OUT_DIM = 9
ENC_DIM = N_LEVELS * F_PER_LEVEL

NC, NS = 2, 16
NW = NC * NS
P = 64
G = P // 16
BLOCKS = 49
PER_W = P * BLOCKS
N_PAD = NW * PER_W

RES = [float(np.floor(BASE_RES * (PER_LEVEL_SCALE ** l))) for l in range(N_LEVELS)]


def _u32(x):
    return jnp.uint32(x)


def _sc_encode(in_flat, ptab, ntab, res_hbm):
    mesh = plsc.VectorSubcoreMesh(core_axis_name="c", subcore_axis_name="s")

    @functools.partial(
        pl.kernel,
        mesh=mesh,
        compiler_params=pltpu.CompilerParams(needs_layout_passes=False),
        out_type=jax.ShapeDtypeStruct((N_PAD * 2 * ENC_DIM,), jnp.float32),
        scratch_types=[
            pltpu.VMEM((7 * P,), jnp.float32),
            pltpu.VMEM((16,), jnp.float32),
            pltpu.VMEM((16, 128), jnp.int32),
            pltpu.VMEM((2048,), jnp.float32),
            pltpu.VMEM((8, 128), jnp.int32),
            pltpu.VMEM((1024,), jnp.float32),
            pltpu.VMEM((P * 2 * ENC_DIM,), jnp.float32),
            pltpu.SemaphoreType.DMA,
            pltpu.SemaphoreType.DMA,
        ],
    )
    def enc(in_hbm, ptab_hbm, ntab_hbm, resl_hbm, out_hbm,
            in_v, res_v, idx_p, rows_p, idx_n, rows_n, feat_v, sem_p, sem_n):
        wid = lax.axis_index("c") * NS + lax.axis_index("s")
        wbase = wid * PER_W
        pltpu.sync_copy(resl_hbm, res_v)
        iota = lax.iota(jnp.int32, 16)
        iotau = lax.iota(jnp.uint32, 16)
        zero16 = jnp.zeros((16,), jnp.int32)
        one16 = jnp.ones((16,), jnp.int32)

        def block_body(blk, carry):
            base = wbase + blk * P
            for d in range(7):
                pltpu.sync_copy(in_hbm.at[pl.ds(d * N_PAD + base, P)],
                                in_v.at[pl.ds(d * P, P)])

            def level_body(l, carry2):
                resv = res_v[...]
                res = lax.gather(
                    resv, jnp.full((16, 1), l, jnp.int32),
                    lax.GatherDimensionNumbers(offset_dims=(),
                                               collapsed_slice_dims=(0,),
                                               start_index_map=(0,)),
                    (1,), mode=lax.GatherScatterMode.PROMISE_IN_BOUNDS)
                loff = l << LOG2_T

                fracs_p = []
                for g in range(G):
                    c0 = g * 16
                    x = in_v[pl.ds(0 * P + c0, 16)] * res
                    y = in_v[pl.ds(1 * P + c0, 16)] * res
                    z = in_v[pl.ds(2 * P + c0, 16)] * res
                    t = in_v[pl.ds(3 * P + c0, 16)] * res
                    xi = x.astype(jnp.int32)
                    yi = y.astype(jnp.int32)
                    zi = z.astype(jnp.int32)
                    ti = t.astype(jnp.int32)
                    fx = x - xi.astype(jnp.float32)
                    fy = y - yi.astype(jnp.float32)
                    fz = z - zi.astype(jnp.float32)
                    ft = t - ti.astype(jnp.float32)
                    fracs_p.append((fx, fy, fz, ft))
                    xu = xi.astype(jnp.uint32)
                    yu = yi.astype(jnp.uint32)
                    zu = zi.astype(jnp.uint32)
                    tu = ti.astype(jnp.uint32)
                    hx = (xu, xu + _u32(1))
                    hy = (yu * _u32(P1), (yu + _u32(1)) * _u32(P1))
                    hz = (zu * _u32(P2), (zu + _u32(1)) * _u32(P2))
                    ht = (tu * _u32(P3), (tu + _u32(1)) * _u32(P3))
                    hxy = [[hx[a] ^ hy[b] for b in range(2)] for a in range(2)]
                    hzt = [[hz[a] ^ ht[b] for b in range(2)] for a in range(2)]
                    for c in range(16):
                        h = hxy[(c >> 3) & 1][(c >> 2) & 1] ^ hzt[(c >> 1) & 1][c & 1]
                        idx = (((h & _u32(MASK)).astype(jnp.int32) + loff) << 1)
                        p0 = g * 256 + c * 16
                        idx_p[p0 >> 7, pl.ds(p0 & 127, 16)] = idx
                        idx_p[8 + (p0 >> 7), pl.ds(p0 & 127, 16)] = idx + 1
                cps_p = [pltpu.async_copy(ptab_hbm.at[idx_p.at[r]], rows_p.at[pl.ds(r * 128, 128)], sem_p)
                         for r in range(16)]

                fracs_n = []
                for g in range(G):
                    c0 = g * 16
                    x = in_v[pl.ds(4 * P + c0, 16)] * res
                    y = in_v[pl.ds(5 * P + c0, 16)] * res
                    z = in_v[pl.ds(6 * P + c0, 16)] * res
                    xi = x.astype(jnp.int32)
                    yi = y.astype(jnp.int32)
                    zi = z.astype(jnp.int32)
                    fx = x - xi.astype(jnp.float32)
                    fy = y - yi.astype(jnp.float32)
                    fz = z - zi.astype(jnp.float32)
                    fracs_n.append((fx, fy, fz))
                    xu = xi.astype(jnp.uint32)
                    yu = yi.astype(jnp.uint32)
                    zu = zi.astype(jnp.uint32)
                    hx = (xu, xu + _u32(1))
                    hy = (yu * _u32(P1), (yu + _u32(1)) * _u32(P1))
                    hz = (zu * _u32(P2), (zu + _u32(1)) * _u32(P2))
                    hxy = [[hx[a] ^ hy[b] for b in range(2)] for a in range(2)]
                    for c in range(8):
                        h = hxy[(c >> 2) & 1][(c >> 1) & 1] ^ hz[c & 1]
                        idx = (((h & _u32(MASK)).astype(jnp.int32) + loff) << 1)
                        p0 = g * 128 + c * 16
                        idx_n[p0 >> 7, pl.ds(p0 & 127, 16)] = idx
                        idx_n[4 + (p0 >> 7), pl.ds(p0 & 127, 16)] = idx + 1
                cps_n = [pltpu.async_copy(ntab_hbm.at[idx_n.at[r]], rows_n.at[pl.ds(r * 128, 128)], sem_n)
                         for r in range(8)]

                for cp in cps_p:
                    cp.wait()
                col0 = jnp.full((16,), 2 * l, jnp.int32)
                col1 = col0 + 1
                for g in range(G):
                    fx, fy, fz, ft = fracs_p[g]
                    wx = (1.0 - fx, fx)
                    wy = (1.0 - fy, fy)
                    wz = (1.0 - fz, fz)
                    wt = (1.0 - ft, ft)
                    acc0 = jnp.zeros((16,), jnp.float32)
                    acc1 = jnp.zeros((16,), jnp.float32)
                    for c in range(16):
                        w = (wx[(c >> 3) & 1] * wy[(c >> 2) & 1]
                             * wz[(c >> 1) & 1] * wt[c & 1])
                        p0 = g * 256 + c * 16
                        f0 = rows_p[pl.ds(p0, 16)]
                        f1 = rows_p[pl.ds(1024 + p0, 16)]
                        acc0 = acc0 + w * f0
                        acc1 = acc1 + w * f1
                    rowv = (iota + g * 16) * (2 * ENC_DIM)
                    plsc.store_scatter(feat_v, [rowv + col0], acc0)
                    plsc.store_scatter(feat_v, [rowv + col1], acc1)

                for cp in cps_n:
                    cp.wait()
                ncol0 = col0 + ENC_DIM
                ncol1 = col1 + ENC_DIM
                for g in range(G):
                    fx, fy, fz = fracs_n[g]
                    wx = (1.0 - fx, fx)
                    wy = (1.0 - fy, fy)
                    wz = (1.0 - fz, fz)
                    acc0 = jnp.zeros((16,), jnp.float32)
                    acc1 = jnp.zeros((16,), jnp.float32)
                    for c in range(8):
                        w = wx[(c >> 2) & 1] * wy[(c >> 1) & 1] * wz[c & 1]
                        p0 = g * 128 + c * 16
                        f0 = rows_n[pl.ds(p0, 16)]
                        f1 = rows_n[pl.ds(512 + p0, 16)]
                        acc0 = acc0 + w * f0
                        acc1 = acc1 + w * f1
                    rowv = (iota + g * 16) * (2 * ENC_DIM)
                    plsc.store_scatter(feat_v, [rowv + ncol0], acc0)
                    plsc.store_scatter(feat_v, [rowv + ncol1], acc1)
                return carry2

            lax.fori_loop(0, N_LEVELS, level_body, 0, unroll=False)
            pltpu.sync_copy(feat_v, out_hbm.at[pl.ds(base * (2 * ENC_DIM), P * 2 * ENC_DIM)])
            return carry

        lax.fori_loop(0, BLOCKS, block_body, 0, unroll=False)

    return enc(in_flat, ptab, ntab, res_hbm)


def _mlp(feat, W1, W2, W3):
    B = 512

    def body(x_ref, w1_ref, w2_ref, w3_ref, o_ref):
        h = jnp.maximum(jnp.dot(x_ref[...], w1_ref[...],
                                preferred_element_type=jnp.float32), 0.0)
        h = jnp.maximum(jnp.dot(h, w2_ref[...],
                                preferred_element_type=jnp.float32), 0.0)
        o_ref[...] = jnp.dot(h, w3_ref[...], preferred_element_type=jnp.float32)

    return pl.pallas_call(
        body,
        grid=(N_PAD // B,),
        in_specs=[
            pl.BlockSpec((B, 2 * ENC_DIM), lambda i: (i, 0)),
            pl.BlockSpec((2 * ENC_DIM, HIDDEN), lambda i: (0, 0)),
            pl.BlockSpec((HIDDEN, HIDDEN), lambda i: (0, 0)),
            pl.BlockSpec((HIDDEN, OUT_DIM), lambda i: (0, 0)),
        ],
        out_specs=pl.BlockSpec((B, OUT_DIM), lambda i: (i, 0)),
        out_shape=jax.ShapeDtypeStruct((N_PAD, OUT_DIM), jnp.float32),
    )(feat, W1, W2, W3)


def kernel(face_centers, time_extended, face_normals, pos_tables, normal_tables, W1, W2, W3):
    n = face_centers.shape[0]
    cols = [face_centers[:, 0], face_centers[:, 1], face_centers[:, 2],
            time_extended[:, 0], face_normals[:, 0], face_normals[:, 1],
            face_normals[:, 2]]
    in_flat = jnp.concatenate(
        [jnp.pad(c, (0, N_PAD - n)) for c in cols]).astype(jnp.float32)
    ptab = pos_tables.reshape(N_LEVELS * T * F_PER_LEVEL)
    ntab = normal_tables.reshape(N_LEVELS * T * F_PER_LEVEL)
    res_hbm = jnp.asarray(RES, dtype=jnp.float32)
    feat = _sc_encode(in_flat, ptab, ntab, res_hbm).reshape(N_PAD, 2 * ENC_DIM)
    out = _mlp(feat, W1, W2, W3)
    return out[:n]


# speedup vs baseline: 1.0009x; 1.0009x over previous
import functools

import jax
import jax.numpy as jnp
import numpy as np
from jax import lax
from jax.experimental import pallas as pl
from jax.experimental.pallas import tpu as pltpu
from jax.experimental.pallas import tpu_sc as plsc

N_LEVELS = 16
F_PER_LEVEL = 2
LOG2_T = 19
T = 1 << LOG2_T
MASK = T - 1
BASE_RES = 16
PER_LEVEL_SCALE = 1.5
P1 = 2654435761
P2 = 805459861
P3 = 3674653429
HIDDEN = 64
OUT_DIM = 9
ENC_DIM = N_LEVELS * F_PER_LEVEL

NC, NS = 2, 16
NW = NC * NS
P = 64
G = P // 16
BLOCKS = 49
PER_W = P * BLOCKS
N_PAD = NW * PER_W

NP_ELEM = P * 16 * 2
NN_ELEM = P * 8 * 2

RES = [float(np.floor(BASE_RES * (PER_LEVEL_SCALE ** l))) for l in range(N_LEVELS)]


def _u32(x):
    return jnp.uint32(x)


def _sc_encode(in_flat, ptab, ntab, res_hbm):
    mesh = plsc.VectorSubcoreMesh(core_axis_name="c", subcore_axis_name="s")

    @functools.partial(
        pl.kernel,
        mesh=mesh,
        compiler_params=pltpu.CompilerParams(needs_layout_passes=False),
        out_type=jax.ShapeDtypeStruct((N_PAD * 2 * ENC_DIM,), jnp.float32),
        scratch_types=[
            pltpu.VMEM((7 * P,), jnp.float32),
            pltpu.VMEM((16,), jnp.float32),
            pltpu.VMEM((NP_ELEM,), jnp.int32),
            pltpu.VMEM((NP_ELEM,), jnp.float32),
            pltpu.VMEM((NN_ELEM,), jnp.int32),
            pltpu.VMEM((NN_ELEM,), jnp.float32),
            pltpu.VMEM((P * 2 * ENC_DIM,), jnp.float32),
            pltpu.SemaphoreType.DMA,
            pltpu.SemaphoreType.DMA,
        ],
    )
    def enc(in_hbm, ptab_hbm, ntab_hbm, resl_hbm, out_hbm,
            in_v, res_v, idx_p, rows_p, idx_n, rows_n, feat_v, sem_p, sem_n):
        wid = lax.axis_index("c") * NS + lax.axis_index("s")
        wbase = wid * PER_W
        pltpu.sync_copy(resl_hbm, res_v)
        iota = lax.iota(jnp.int32, 16)

        def block_body(blk, carry):
            base = wbase + blk * P
            for d in range(7):
                pltpu.sync_copy(in_hbm.at[pl.ds(d * N_PAD + base, P)],
                                in_v.at[pl.ds(d * P, P)])

            def level_body(l, carry2):
                resv = res_v[...]
                res = lax.gather(
                    resv, jnp.full((16, 1), l, jnp.int32),
                    lax.GatherDimensionNumbers(offset_dims=(),
                                               collapsed_slice_dims=(0,),
                                               start_index_map=(0,)),
                    (1,), mode=lax.GatherScatterMode.PROMISE_IN_BOUNDS)
                loff = l << LOG2_T

                fracs_p = []
                for g in range(G):
                    c0 = g * 16
                    x = in_v[pl.ds(0 * P + c0, 16)] * res
                    y = in_v[pl.ds(1 * P + c0, 16)] * res
                    z = in_v[pl.ds(2 * P + c0, 16)] * res
                    t = in_v[pl.ds(3 * P + c0, 16)] * res
                    xi = x.astype(jnp.int32)
                    yi = y.astype(jnp.int32)
                    zi = z.astype(jnp.int32)
                    ti = t.astype(jnp.int32)
                    fx = x - xi.astype(jnp.float32)
                    fy = y - yi.astype(jnp.float32)
                    fz = z - zi.astype(jnp.float32)
                    ft = t - ti.astype(jnp.float32)
                    fracs_p.append((fx, fy, fz, ft))
                    xu = xi.astype(jnp.uint32)
                    yu = yi.astype(jnp.uint32)
                    zu = zi.astype(jnp.uint32)
                    tu = ti.astype(jnp.uint32)
                    hx = (xu, xu + _u32(1))
                    hy = (yu * _u32(P1), (yu + _u32(1)) * _u32(P1))
                    hz = (zu * _u32(P2), (zu + _u32(1)) * _u32(P2))
                    ht = (tu * _u32(P3), (tu + _u32(1)) * _u32(P3))
                    hxy = [[hx[a] ^ hy[b] for b in range(2)] for a in range(2)]
                    hzt = [[hz[a] ^ ht[b] for b in range(2)] for a in range(2)]
                    for c in range(16):
                        h = hxy[(c >> 3) & 1][(c >> 2) & 1] ^ hzt[(c >> 1) & 1][c & 1]
                        idx = (((h & _u32(MASK)).astype(jnp.int32) + loff) << 1)
                        q0 = g * 256 + c * 16
                        idx_p[pl.ds(q0, 16)] = idx
                        idx_p[pl.ds(1024 + q0, 16)] = idx + 1
                cp_p = pltpu.async_copy(ptab_hbm.at[idx_p], rows_p, sem_p)

                fracs_n = []
                for g in range(G):
                    c0 = g * 16
                    x = in_v[pl.ds(4 * P + c0, 16)] * res
                    y = in_v[pl.ds(5 * P + c0, 16)] * res
                    z = in_v[pl.ds(6 * P + c0, 16)] * res
                    xi = x.astype(jnp.int32)
                    yi = y.astype(jnp.int32)
                    zi = z.astype(jnp.int32)
                    fx = x - xi.astype(jnp.float32)
                    fy = y - yi.astype(jnp.float32)
                    fz = z - zi.astype(jnp.float32)
                    fracs_n.append((fx, fy, fz))
                    xu = xi.astype(jnp.uint32)
                    yu = yi.astype(jnp.uint32)
                    zu = zi.astype(jnp.uint32)
                    hx = (xu, xu + _u32(1))
                    hy = (yu * _u32(P1), (yu + _u32(1)) * _u32(P1))
                    hz = (zu * _u32(P2), (zu + _u32(1)) * _u32(P2))
                    hxy = [[hx[a] ^ hy[b] for b in range(2)] for a in range(2)]
                    for c in range(8):
                        h = hxy[(c >> 2) & 1][(c >> 1) & 1] ^ hz[c & 1]
                        idx = (((h & _u32(MASK)).astype(jnp.int32) + loff) << 1)
                        q0 = g * 128 + c * 16
                        idx_n[pl.ds(q0, 16)] = idx
                        idx_n[pl.ds(512 + q0, 16)] = idx + 1
                cp_n = pltpu.async_copy(ntab_hbm.at[idx_n], rows_n, sem_n)

                cp_p.wait()
                col0 = jnp.full((16,), 2 * l, jnp.int32)
                col1 = col0 + 1
                for g in range(G):
                    fx, fy, fz, ft = fracs_p[g]
                    wx = (1.0 - fx, fx)
                    wy = (1.0 - fy, fy)
                    wz = (1.0 - fz, fz)
                    wt = (1.0 - ft, ft)
                    acc0 = jnp.zeros((16,), jnp.float32)
                    acc1 = jnp.zeros((16,), jnp.float32)
                    for c in range(16):
                        w = (wx[(c >> 3) & 1] * wy[(c >> 2) & 1]
                             * wz[(c >> 1) & 1] * wt[c & 1])
                        q0 = g * 256 + c * 16
                        f0 = rows_p[pl.ds(q0, 16)]
                        f1 = rows_p[pl.ds(1024 + q0, 16)]
                        acc0 = acc0 + w * f0
                        acc1 = acc1 + w * f1
                    rowv = (iota + g * 16) * (2 * ENC_DIM)
                    plsc.store_scatter(feat_v, [rowv + col0], acc0)
                    plsc.store_scatter(feat_v, [rowv + col1], acc1)

                cp_n.wait()
                ncol0 = col0 + ENC_DIM
                ncol1 = col1 + ENC_DIM
                for g in range(G):
                    fx, fy, fz = fracs_n[g]
                    wx = (1.0 - fx, fx)
                    wy = (1.0 - fy, fy)
                    wz = (1.0 - fz, fz)
                    acc0 = jnp.zeros((16,), jnp.float32)
                    acc1 = jnp.zeros((16,), jnp.float32)
                    for c in range(8):
                        w = wx[(c >> 2) & 1] * wy[(c >> 1) & 1] * wz[c & 1]
                        q0 = g * 128 + c * 16
                        f0 = rows_n[pl.ds(q0, 16)]
                        f1 = rows_n[pl.ds(512 + q0, 16)]
                        acc0 = acc0 + w * f0
                        acc1 = acc1 + w * f1
                    rowv = (iota + g * 16) * (2 * ENC_DIM)
                    plsc.store_scatter(feat_v, [rowv + ncol0], acc0)
                    plsc.store_scatter(feat_v, [rowv + ncol1], acc1)
                return carry2

            lax.fori_loop(0, N_LEVELS, level_body, 0, unroll=False)
            pltpu.sync_copy(feat_v, out_hbm.at[pl.ds(base * (2 * ENC_DIM), P * 2 * ENC_DIM)])
            return carry

        lax.fori_loop(0, BLOCKS, block_body, 0, unroll=False)

    return enc(in_flat, ptab, ntab, res_hbm)


def _mlp(feat, W1, W2, W3):
    B = 512

    def body(x_ref, w1_ref, w2_ref, w3_ref, o_ref):
        h = jnp.maximum(jnp.dot(x_ref[...], w1_ref[...],
                                preferred_element_type=jnp.float32), 0.0)
        h = jnp.maximum(jnp.dot(h, w2_ref[...],
                                preferred_element_type=jnp.float32), 0.0)
        o_ref[...] = jnp.dot(h, w3_ref[...], preferred_element_type=jnp.float32)

    return pl.pallas_call(
        body,
        grid=(N_PAD // B,),
        in_specs=[
            pl.BlockSpec((B, 2 * ENC_DIM), lambda i: (i, 0)),
            pl.BlockSpec((2 * ENC_DIM, HIDDEN), lambda i: (0, 0)),
            pl.BlockSpec((HIDDEN, HIDDEN), lambda i: (0, 0)),
            pl.BlockSpec((HIDDEN, OUT_DIM), lambda i: (0, 0)),
        ],
        out_specs=pl.BlockSpec((B, OUT_DIM), lambda i: (i, 0)),
        out_shape=jax.ShapeDtypeStruct((N_PAD, OUT_DIM), jnp.float32),
    )(feat, W1, W2, W3)


def kernel(face_centers, time_extended, face_normals, pos_tables, normal_tables, W1, W2, W3):
    n = face_centers.shape[0]
    cols = [face_centers[:, 0], face_centers[:, 1], face_centers[:, 2],
            time_extended[:, 0], face_normals[:, 0], face_normals[:, 1],
            face_normals[:, 2]]
    in_flat = jnp.concatenate(
        [jnp.pad(c, (0, N_PAD - n)) for c in cols]).astype(jnp.float32)
    ptab = pos_tables.reshape(N_LEVELS * T * F_PER_LEVEL)
    ntab = normal_tables.reshape(N_LEVELS * T * F_PER_LEVEL)
    res_hbm = jnp.asarray(RES, dtype=jnp.float32)
    feat = _sc_encode(in_flat, ptab, ntab, res_hbm).reshape(N_PAD, 2 * ENC_DIM)
    out = _mlp(feat, W1, W2, W3)
    return out[:n]


# speedup vs baseline: 6.6336x; 6.6276x over previous
import functools

import jax
import jax.numpy as jnp
import numpy as np
from jax import lax
from jax.experimental import pallas as pl
from jax.experimental.pallas import tpu as pltpu
from jax.experimental.pallas import tpu_sc as plsc

N_LEVELS = 16
F_PER_LEVEL = 2
LOG2_T = 19
T = 1 << LOG2_T
MASK = T - 1
BASE_RES = 16
PER_LEVEL_SCALE = 1.5
P1 = 2654435761
P2 = 805459861
P3 = 3674653429
HIDDEN = 64
OUT_DIM = 9
ENC_DIM = N_LEVELS * F_PER_LEVEL

NC, NS = 2, 16
NW = NC * NS
P = 64
G = P // 16
BLOCKS = 49
PER_W = P * BLOCKS
N_PAD = NW * PER_W

NP_ELEM = P * 16
NN_ELEM = P * 8

RES = [float(np.floor(BASE_RES * (PER_LEVEL_SCALE ** l))) for l in range(N_LEVELS)]


def _u32(x):
    return jnp.uint32(x)


def _sc_encode(in_flat, p0t, p1t, n0t, n1t, res_hbm):
    mesh = plsc.VectorSubcoreMesh(core_axis_name="c", subcore_axis_name="s")

    @functools.partial(
        pl.kernel,
        mesh=mesh,
        compiler_params=pltpu.CompilerParams(needs_layout_passes=False),
        out_type=jax.ShapeDtypeStruct((N_PAD * 2 * ENC_DIM,), jnp.float32),
        scratch_types=[
            pltpu.VMEM((7 * P,), jnp.float32),
            pltpu.VMEM((16,), jnp.float32),
            pltpu.VMEM((NP_ELEM,), jnp.int32),
            pltpu.VMEM((NP_ELEM,), jnp.float32),
            pltpu.VMEM((NP_ELEM,), jnp.float32),
            pltpu.VMEM((NN_ELEM,), jnp.int32),
            pltpu.VMEM((NN_ELEM,), jnp.float32),
            pltpu.VMEM((NN_ELEM,), jnp.float32),
            pltpu.VMEM((P * 2 * ENC_DIM,), jnp.float32),
            pltpu.SemaphoreType.DMA,
            pltpu.SemaphoreType.DMA,
        ],
    )
    def enc(in_hbm, p0_hbm, p1_hbm, n0_hbm, n1_hbm, resl_hbm, out_hbm,
            in_v, res_v, idx_p, rows_p0, rows_p1, idx_n, rows_n0, rows_n1,
            feat_v, sem_p, sem_n):
        wid = lax.axis_index("c") * NS + lax.axis_index("s")
        wbase = wid * PER_W
        pltpu.sync_copy(resl_hbm, res_v)
        iota = lax.iota(jnp.int32, 16)

        def block_body(blk, carry):
            base = wbase + blk * P
            for d in range(7):
                pltpu.sync_copy(in_hbm.at[pl.ds(d * N_PAD + base, P)],
                                in_v.at[pl.ds(d * P, P)])

            def level_body(l, carry2):
                resv = res_v[...]
                res = lax.gather(
                    resv, jnp.full((16, 1), l, jnp.int32),
                    lax.GatherDimensionNumbers(offset_dims=(),
                                               collapsed_slice_dims=(0,),
                                               start_index_map=(0,)),
                    (1,), mode=lax.GatherScatterMode.PROMISE_IN_BOUNDS)
                loff = l << LOG2_T

                fracs_p = []
                for g in range(G):
                    c0 = g * 16
                    x = in_v[pl.ds(0 * P + c0, 16)] * res
                    y = in_v[pl.ds(1 * P + c0, 16)] * res
                    z = in_v[pl.ds(2 * P + c0, 16)] * res
                    t = in_v[pl.ds(3 * P + c0, 16)] * res
                    xi = x.astype(jnp.int32)
                    yi = y.astype(jnp.int32)
                    zi = z.astype(jnp.int32)
                    ti = t.astype(jnp.int32)
                    fx = x - xi.astype(jnp.float32)
                    fy = y - yi.astype(jnp.float32)
                    fz = z - zi.astype(jnp.float32)
                    ft = t - ti.astype(jnp.float32)
                    fracs_p.append((fx, fy, fz, ft))
                    xu = xi.astype(jnp.uint32)
                    yu = yi.astype(jnp.uint32)
                    zu = zi.astype(jnp.uint32)
                    tu = ti.astype(jnp.uint32)
                    hx = (xu, xu + _u32(1))
                    hy = (yu * _u32(P1), (yu + _u32(1)) * _u32(P1))
                    hz = (zu * _u32(P2), (zu + _u32(1)) * _u32(P2))
                    ht = (tu * _u32(P3), (tu + _u32(1)) * _u32(P3))
                    hxy = [[hx[a] ^ hy[b] for b in range(2)] for a in range(2)]
                    hzt = [[hz[a] ^ ht[b] for b in range(2)] for a in range(2)]
                    for c in range(16):
                        h = hxy[(c >> 3) & 1][(c >> 2) & 1] ^ hzt[(c >> 1) & 1][c & 1]
                        idx = (h & _u32(MASK)).astype(jnp.int32) + loff
                        q0 = g * 256 + c * 16
                        idx_p[pl.ds(q0, 16)] = idx
                cps_p = [pltpu.async_copy(p0_hbm.at[idx_p], rows_p0, sem_p),
                         pltpu.async_copy(p1_hbm.at[idx_p], rows_p1, sem_p)]

                fracs_n = []
                for g in range(G):
                    c0 = g * 16
                    x = in_v[pl.ds(4 * P + c0, 16)] * res
                    y = in_v[pl.ds(5 * P + c0, 16)] * res
                    z = in_v[pl.ds(6 * P + c0, 16)] * res
                    xi = x.astype(jnp.int32)
                    yi = y.astype(jnp.int32)
                    zi = z.astype(jnp.int32)
                    fx = x - xi.astype(jnp.float32)
                    fy = y - yi.astype(jnp.float32)
                    fz = z - zi.astype(jnp.float32)
                    fracs_n.append((fx, fy, fz))
                    xu = xi.astype(jnp.uint32)
                    yu = yi.astype(jnp.uint32)
                    zu = zi.astype(jnp.uint32)
                    hx = (xu, xu + _u32(1))
                    hy = (yu * _u32(P1), (yu + _u32(1)) * _u32(P1))
                    hz = (zu * _u32(P2), (zu + _u32(1)) * _u32(P2))
                    hxy = [[hx[a] ^ hy[b] for b in range(2)] for a in range(2)]
                    for c in range(8):
                        h = hxy[(c >> 2) & 1][(c >> 1) & 1] ^ hz[c & 1]
                        idx = (h & _u32(MASK)).astype(jnp.int32) + loff
                        q0 = g * 128 + c * 16
                        idx_n[pl.ds(q0, 16)] = idx
                cps_n = [pltpu.async_copy(n0_hbm.at[idx_n], rows_n0, sem_n),
                         pltpu.async_copy(n1_hbm.at[idx_n], rows_n1, sem_n)]

                for cp in cps_p:
                    cp.wait()
                col0 = jnp.full((16,), 2 * l, jnp.int32)
                col1 = col0 + 1
                for g in range(G):
                    fx, fy, fz, ft = fracs_p[g]
                    wx = (1.0 - fx, fx)
                    wy = (1.0 - fy, fy)
                    wz = (1.0 - fz, fz)
                    wt = (1.0 - ft, ft)
                    acc0 = jnp.zeros((16,), jnp.float32)
                    acc1 = jnp.zeros((16,), jnp.float32)
                    for c in range(16):
                        w = (wx[(c >> 3) & 1] * wy[(c >> 2) & 1]
                             * wz[(c >> 1) & 1] * wt[c & 1])
                        q0 = g * 256 + c * 16
                        f0 = rows_p0[pl.ds(q0, 16)]
                        f1 = rows_p1[pl.ds(q0, 16)]
                        acc0 = acc0 + w * f0
                        acc1 = acc1 + w * f1
                    rowv = (iota + g * 16) * (2 * ENC_DIM)
                    plsc.store_scatter(feat_v, [rowv + col0], acc0)
                    plsc.store_scatter(feat_v, [rowv + col1], acc1)

                for cp in cps_n:
                    cp.wait()
                ncol0 = col0 + ENC_DIM
                ncol1 = col1 + ENC_DIM
                for g in range(G):
                    fx, fy, fz = fracs_n[g]
                    wx = (1.0 - fx, fx)
                    wy = (1.0 - fy, fy)
                    wz = (1.0 - fz, fz)
                    acc0 = jnp.zeros((16,), jnp.float32)
                    acc1 = jnp.zeros((16,), jnp.float32)
                    for c in range(8):
                        w = wx[(c >> 2) & 1] * wy[(c >> 1) & 1] * wz[c & 1]
                        q0 = g * 128 + c * 16
                        f0 = rows_n0[pl.ds(q0, 16)]
                        f1 = rows_n1[pl.ds(q0, 16)]
                        acc0 = acc0 + w * f0
                        acc1 = acc1 + w * f1
                    rowv = (iota + g * 16) * (2 * ENC_DIM)
                    plsc.store_scatter(feat_v, [rowv + ncol0], acc0)
                    plsc.store_scatter(feat_v, [rowv + ncol1], acc1)
                return carry2

            lax.fori_loop(0, N_LEVELS, level_body, 0, unroll=False)
            pltpu.sync_copy(feat_v, out_hbm.at[pl.ds(base * (2 * ENC_DIM), P * 2 * ENC_DIM)])
            return carry

        lax.fori_loop(0, BLOCKS, block_body, 0, unroll=False)

    return enc(in_flat, p0t, p1t, n0t, n1t, res_hbm)


def _mlp(feat, W1, W2, W3):
    B = 512

    def body(x_ref, w1_ref, w2_ref, w3_ref, o_ref):
        h = jnp.maximum(jnp.dot(x_ref[...], w1_ref[...],
                                preferred_element_type=jnp.float32), 0.0)
        h = jnp.maximum(jnp.dot(h, w2_ref[...],
                                preferred_element_type=jnp.float32), 0.0)
        o_ref[...] = jnp.dot(h, w3_ref[...], preferred_element_type=jnp.float32)

    return pl.pallas_call(
        body,
        grid=(N_PAD // B,),
        in_specs=[
            pl.BlockSpec((B, 2 * ENC_DIM), lambda i: (i, 0)),
            pl.BlockSpec((2 * ENC_DIM, HIDDEN), lambda i: (0, 0)),
            pl.BlockSpec((HIDDEN, HIDDEN), lambda i: (0, 0)),
            pl.BlockSpec((HIDDEN, OUT_DIM), lambda i: (0, 0)),
        ],
        out_specs=pl.BlockSpec((B, OUT_DIM), lambda i: (i, 0)),
        out_shape=jax.ShapeDtypeStruct((N_PAD, OUT_DIM), jnp.float32),
    )(feat, W1, W2, W3)


def kernel(face_centers, time_extended, face_normals, pos_tables, normal_tables, W1, W2, W3):
    n = face_centers.shape[0]
    cols = [face_centers[:, 0], face_centers[:, 1], face_centers[:, 2],
            time_extended[:, 0], face_normals[:, 0], face_normals[:, 1],
            face_normals[:, 2]]
    in_flat = jnp.concatenate(
        [jnp.pad(c, (0, N_PAD - n)) for c in cols]).astype(jnp.float32)
    p0t = pos_tables[:, :, 0].reshape(N_LEVELS * T)
    p1t = pos_tables[:, :, 1].reshape(N_LEVELS * T)
    n0t = normal_tables[:, :, 0].reshape(N_LEVELS * T)
    n1t = normal_tables[:, :, 1].reshape(N_LEVELS * T)
    res_hbm = jnp.asarray(RES, dtype=jnp.float32)
    feat = _sc_encode(in_flat, p0t, p1t, n0t, n1t, res_hbm).reshape(N_PAD, 2 * ENC_DIM)
    out = _mlp(feat, W1, W2, W3)
    return out[:n]
